# baseline (speedup 1.0000x reference)
import functools

import jax
import jax.numpy as jnp
from jax import lax
from jax.experimental import pallas as pl
from jax.experimental.pallas import tpu as pltpu
from jax.experimental.pallas import tpu_sc as plsc

B = 4
T = 2048
D = 1024

_info = plsc.get_sparse_core_info()
_NC, _NS, _L = _info.num_cores, _info.num_subcores, _info.num_lanes
_NW = _NC * _NS
_NTOK = B * T
_PER_W = _NTOK // _NW
_C = 32
_NCHUNK = _PER_W // _C
_VPR = D // _L

_mesh = plsc.VectorSubcoreMesh(core_axis_name="c", subcore_axis_name="s")


@functools.partial(
    pl.kernel,
    mesh=_mesh,
    out_type=jax.ShapeDtypeStruct((_NTOK, D), jnp.float32),
    scratch_types=[
        pltpu.VMEM((_PER_W,), jnp.int32),
        pltpu.VMEM((_C, D), jnp.float32),
        pltpu.VMEM((_C, D), jnp.float32),
        pltpu.SemaphoreType.DMA,
        pltpu.SemaphoreType.DMA,
    ],
)
def _emb_kernel(x_hbm, emb_hbm, pos_hbm, out_hbm, idx_v, rows_v, pos_v,
                gsem, psem):
    wid = lax.axis_index("s") * _NC + lax.axis_index("c")
    base = wid * _PER_W
    t0 = base % T
    pltpu.sync_copy(x_hbm.at[pl.ds(base, _PER_W)], idx_v)

    for i in range(_NCHUNK):
        off = i * _C
        gcp = pltpu.async_copy(
            emb_hbm.at[idx_v.at[pl.ds(off, _C)]], rows_v, gsem)
        pcp = pltpu.async_copy(
            pos_hbm.at[pl.ds(t0 + off, _C)], pos_v, psem)
        gcp.wait()
        pcp.wait()

        def _add_row(r, carry):
            for v in range(_VPR):
                sl = pl.ds(v * _L, _L)
                plsc.addupdate(rows_v.at[r, sl], pos_v[r, sl])
            return carry

        lax.fori_loop(0, _C, _add_row, 0)
        pltpu.sync_copy(rows_v, out_hbm.at[pl.ds(base + off, _C)])


def kernel(x, emb_table, pos_table):
    x_flat = x.reshape(-1).astype(jnp.int32)
    out = _emb_kernel(x_flat, emb_table, pos_table)
    return out.reshape(B, T, D)


# speedup vs baseline: 1.1928x; 1.1928x over previous
import functools

import jax
import jax.numpy as jnp
from jax import lax
from jax.experimental import pallas as pl
from jax.experimental.pallas import tpu as pltpu
from jax.experimental.pallas import tpu_sc as plsc

B = 4
T = 2048
D = 1024

_info = plsc.get_sparse_core_info()
_NC, _NS, _L = _info.num_cores, _info.num_subcores, _info.num_lanes
_NW = _NC * _NS
_NTOK = B * T
_TW = T // _NW
_CR = 16
_CPB = _TW // _CR
_NCH = B * _CPB
_NBUF = 3
_VPR = D // _L

_mesh = plsc.VectorSubcoreMesh(core_axis_name="c", subcore_axis_name="s")


@functools.partial(
    pl.kernel,
    mesh=_mesh,
    out_type=jax.ShapeDtypeStruct((_NTOK, D), jnp.float32),
    scratch_types=[
        pltpu.VMEM((B * _TW,), jnp.int32),
        pltpu.VMEM((_TW, D), jnp.float32),
    ] + [pltpu.VMEM((_CR, D), jnp.float32) for _ in range(_NBUF)]
      + [pltpu.SemaphoreType.DMA for _ in range(2 * _NBUF + 2)],
)
def _emb_kernel(x_hbm, emb_hbm, pos_hbm, out_hbm, idx_v, pos_v,
                buf0, buf1, buf2, g0, g1, g2, w0, w1, w2, psem, isem):
    bufs = (buf0, buf1, buf2)
    gsems = (g0, g1, g2)
    wsems = (w0, w1, w2)
    wid = lax.axis_index("s") * _NC + lax.axis_index("c")
    t0 = wid * _TW

    icps = [pltpu.async_copy(x_hbm.at[pl.ds(b * T + t0, _TW)],
                             idx_v.at[pl.ds(b * _TW, _TW)], isem)
            for b in range(B)]
    pcp = pltpu.async_copy(pos_hbm.at[pl.ds(t0, _TW)], pos_v, psem)
    for cp in icps:
        cp.wait()

    def start_gather(j, p):
        b, c = j >> 2, j & 3
        return pltpu.async_copy(
            emb_hbm.at[idx_v.at[pl.ds(b * _TW + _CR * c, _CR)]],
            bufs[p], gsems[p])

    gcps = [start_gather(0, 0), start_gather(1, 1)]
    wcps = [None, None, None]

    pcp.wait()
    for j in range(_NCH):
        if j + 2 < _NCH:
            pn = (j + 2) % _NBUF
            if wcps[pn] is not None:
                wcps[pn].wait()
                wcps[pn] = None
            gcps.append(start_gather(j + 2, pn))
        p = j % _NBUF
        b, c = j >> 2, j & 3
        gcps[j].wait()

        def add_body(r, carry):
            for v in range(_VPR):
                sl = pl.ds(v * _L, _L)
                plsc.addupdate(bufs[p].at[r, sl], pos_v[_CR * c + r, sl])
            return carry

        lax.fori_loop(0, _CR, add_body, 0)

        wcps[p] = pltpu.async_copy(
            bufs[p], out_hbm.at[pl.ds(b * T + t0 + _CR * c, _CR)], wsems[p])
    for cp in wcps:
        if cp is not None:
            cp.wait()


def kernel(x, emb_table, pos_table):
    x_flat = x.reshape(-1).astype(jnp.int32)
    out = _emb_kernel(x_flat, emb_table, pos_table)
    return out.reshape(B, T, D)
